# Initial kernel scaffold; baseline (speedup 1.0000x reference)
import jax
import jax.numpy as jnp
from jax.experimental import pallas as pl


def kernel(x, W):
    raise NotImplementedError("write your pallas kernel here")


# speedup vs baseline: 1.0928x; 1.0928x over previous
import functools

import jax
import jax.numpy as jnp
from jax import lax
from jax.experimental import pallas as pl
from jax.experimental.pallas import tpu as pltpu
from jax.experimental.pallas import tpu_sc as plsc

_INFO = plsc.get_sparse_core_info()
_NC = _INFO.num_cores
_NS = _INFO.num_subcores
_NW = _NC * _NS


@functools.partial(jax.jit, static_argnums=(2, 3, 4))
def _sc_gather(table, idx, B, D, chunk):
    b_per_w = B // _NW
    n_chunks = b_per_w // chunk
    mesh = plsc.VectorSubcoreMesh(core_axis_name="c", subcore_axis_name="s")

    @functools.partial(
        pl.kernel,
        mesh=mesh,
        out_type=jax.ShapeDtypeStruct((B, D), jnp.float32),
        scratch_types=[
            pltpu.VMEM((chunk,), jnp.int32),
            pltpu.VMEM((chunk, D), jnp.float32),
            pltpu.SemaphoreType.DMA,
        ],
        compiler_params=pltpu.CompilerParams(use_tc_tiling_on_sc=False),
    )
    def k(table_hbm, idx_hbm, out_hbm, idx_v, rows_v, sem):
        wid = lax.axis_index("s") * _NC + lax.axis_index("c")
        base = wid * b_per_w

        def body(i, carry):
            off = base + i * chunk
            pltpu.sync_copy(idx_hbm.at[pl.ds(off, chunk)], idx_v)
            pltpu.async_copy(table_hbm.at[idx_v], rows_v, sem).wait()
            pltpu.sync_copy(rows_v, out_hbm.at[pl.ds(off, chunk)])
            return carry

        lax.fori_loop(0, n_chunks, body, 0)

    return k(table, idx)


def kernel(x, W):
    B = x.shape[0] * x.shape[1]
    D = W.shape[1]
    flat = x.reshape(B)
    out = _sc_gather(W, flat, B, D, 1024)
    return out.reshape(x.shape[0], x.shape[1], D)


# speedup vs baseline: 1.1114x; 1.0170x over previous
import functools

import jax
import jax.numpy as jnp
from jax import lax
from jax.experimental import pallas as pl
from jax.experimental.pallas import tpu as pltpu
from jax.experimental.pallas import tpu_sc as plsc

_INFO = plsc.get_sparse_core_info()
_NC = _INFO.num_cores
_NS = _INFO.num_subcores
_NW = _NC * _NS

_NBUF = 3


@functools.partial(jax.jit, static_argnums=(2, 3, 4))
def _sc_gather(table, idx, B, D, chunk):
    b_per_w = B // _NW
    n_chunks = b_per_w // chunk
    mesh = plsc.VectorSubcoreMesh(core_axis_name="c", subcore_axis_name="s")

    @functools.partial(
        pl.kernel,
        mesh=mesh,
        out_type=jax.ShapeDtypeStruct((B, D), jnp.float32),
        scratch_types=[
            pltpu.VMEM((b_per_w,), jnp.int32),
            [pltpu.VMEM((chunk, D), jnp.float32) for _ in range(_NBUF)],
            [pltpu.SemaphoreType.DMA for _ in range(_NBUF)],
            [pltpu.SemaphoreType.DMA for _ in range(_NBUF)],
        ],
        compiler_params=pltpu.CompilerParams(use_tc_tiling_on_sc=False),
    )
    def k(table_hbm, idx_hbm, out_hbm, idx_all, rows, sem_g, sem_s):
        wid = lax.axis_index("s") * _NC + lax.axis_index("c")
        base = wid * b_per_w

        pltpu.sync_copy(idx_hbm.at[pl.ds(base, b_per_w)], idx_all)

        def gather(i, b):
            return pltpu.async_copy(
                table_hbm.at[idx_all.at[pl.ds(i * chunk, chunk)]],
                rows[b], sem_g[b])

        def scatter(i, b):
            return pltpu.async_copy(
                rows[b], out_hbm.at[pl.ds(base + i * chunk, chunk)],
                sem_s[b])

        gdesc = [None] * _NBUF
        sdesc = [None] * _NBUF
        s_waited = [True] * _NBUF

        for i in range(min(2, n_chunks)):
            gdesc[i % _NBUF] = gather(i, i % _NBUF)

        for i in range(n_chunks):
            b = i % _NBUF
            gdesc[b].wait()
            sdesc[b] = scatter(i, b)
            s_waited[b] = False
            ni = i + 2
            if ni < n_chunks:
                nb = ni % _NBUF
                if not s_waited[nb]:
                    sdesc[nb].wait()
                    s_waited[nb] = True
                gdesc[nb] = gather(ni, nb)

        for b in range(_NBUF):
            if not s_waited[b]:
                sdesc[b].wait()

    return k(table, idx)


def kernel(x, W):
    B = x.shape[0] * x.shape[1]
    D = W.shape[1]
    flat = x.reshape(B)
    out = _sc_gather(W, flat, B, D, 1024)
    return out.reshape(x.shape[0], x.shape[1], D)


# speedup vs baseline: 1.4753x; 1.3274x over previous
import functools

import jax
import jax.numpy as jnp
from jax import lax
from jax.experimental import pallas as pl
from jax.experimental.pallas import tpu as pltpu
from jax.experimental.pallas import tpu_sc as plsc

_INFO = plsc.get_sparse_core_info()
_NC = _INFO.num_cores
_NS = _INFO.num_subcores
_NW = _NC * _NS

_V = 1000000
_D = 32
_CA = 512
_NFULL = _V // _CA
_TAIL = _V - _NFULL * _CA
_B = 16384
_H = 50
_CB = _B // _NW


def _iota16():
    return lax.iota(jnp.int32, 16)


@jax.jit
def _transpose_table(wt, tail1d):
    mesh = plsc.VectorSubcoreMesh(core_axis_name="c", subcore_axis_name="s")

    @functools.partial(
        pl.kernel,
        mesh=mesh,
        out_type=jax.ShapeDtypeStruct((_V * _D,), jnp.float32),
        scratch_types=[
            [pltpu.VMEM((_D, _CA), jnp.float32) for _ in range(2)],
            [pltpu.VMEM((_CA * _D,), jnp.float32) for _ in range(2)],
            [pltpu.SemaphoreType.DMA for _ in range(2)],
            [pltpu.SemaphoreType.DMA for _ in range(2)],
        ],
        compiler_params=pltpu.CompilerParams(needs_layout_passes=False),
    )
    def k(wt_hbm, tail_hbm, w1d_hbm, inb, outb, sem_i, sem_o):
        wid = lax.axis_index("s") * _NC + lax.axis_index("c")
        nw = (_NFULL - 1 - wid) // _NW + 1
        iota32 = _iota16() * _D

        def chunk_of(j):
            return wid + _NW * j

        def in_copy(j, p):
            v0 = chunk_of(j) * _CA
            return pltpu.make_async_copy(
                wt_hbm.at[:, pl.ds(v0, _CA)], inb[p], sem_i[p])

        def out_copy(j, p):
            o0 = chunk_of(j) * _CA * _D
            return pltpu.make_async_copy(
                outb[p], w1d_hbm.at[pl.ds(o0, _CA * _D)], sem_o[p])

        in_copy(0, 0).start()
        in_copy(1, 1).start()

        def transpose_chunk(p):
            @pl.loop(0, _CA // 16)
            def g_loop(g):
                base = g * (16 * _D)
                for e in range(_D):
                    vals = inb[p][e, pl.ds(g * 16, 16)]
                    plsc.store_scatter(outb[p], [iota32 + (base + e)], vals)

        @pl.loop(0, 31)
        def j_loop(i):
            for p in range(2):
                j = 2 * i + p

                @pl.when(j < nw)
                def _():
                    in_copy(j, p).wait()

                    @pl.when(j >= 2)
                    def _():
                        out_copy(j - 2, p).wait()

                    transpose_chunk(p)
                    out_copy(j, p).start()

                    @pl.when(j + 2 < nw)
                    def _():
                        in_copy(j + 2, p).start()

        out_copy(nw - 2, 0).wait()
        out_copy(nw - 1, 1).wait()

        @pl.when(wid == 1)
        def _():
            pltpu.sync_copy(tail_hbm, outb[0].at[pl.ds(0, _TAIL * _D)])
            pltpu.sync_copy(outb[0].at[pl.ds(0, _TAIL * _D)],
                            w1d_hbm.at[pl.ds(_NFULL * _CA * _D, _TAIL * _D)])

    return k(wt, tail1d)


@jax.jit
def _gather_to_native(w2d, xt):
    mesh = plsc.VectorSubcoreMesh(core_axis_name="c", subcore_axis_name="s")

    @functools.partial(
        pl.kernel,
        mesh=mesh,
        out_type=jax.ShapeDtypeStruct((_H, _D // 8, (_B // 128) * 8 * 128),
                                      jnp.float32),
        scratch_types=[
            pltpu.VMEM((_H, _CB), jnp.int32),
            [pltpu.VMEM((_CB, _D), jnp.float32) for _ in range(2)],
            [pltpu.VMEM((_D // 8, (_CB // 128) * 8 * 128), jnp.float32)
             for _ in range(2)],
            [pltpu.SemaphoreType.DMA for _ in range(2)],
            [pltpu.SemaphoreType.DMA for _ in range(2)],
        ],
        compiler_params=pltpu.CompilerParams(use_tc_tiling_on_sc=False,
                                             needs_layout_passes=False),
    )
    def k(w_hbm, xt_hbm, out_hbm, idx_all, rows, slab, sem_g, sem_s):
        wid = lax.axis_index("s") * _NC + lax.axis_index("c")
        b0 = wid * _CB
        iota = _iota16()

        pltpu.sync_copy(xt_hbm.at[:, pl.ds(b0, _CB)], idx_all)

        def gather(h, p):
            return pltpu.make_async_copy(
                w_hbm.at[idx_all.at[h]], rows[p], sem_g[p])

        def out_copies(h, p):
            return [
                pltpu.make_async_copy(
                    slab[p].at[ti],
                    out_hbm.at[h, ti, pl.ds(wid * (_CB // 128) * 1024,
                                            (_CB // 128) * 1024)],
                    sem_s[p])
                for ti in range(_D // 8)
            ]

        gather(0, 0).start()
        gather(1, 1).start()

        def assemble(p):
            @pl.loop(0, _CB // 16)
            def g_loop(g):
                rowvec = iota + g * 16
                gofs = (g // 8) * 1024 + (g % 8) * 16
                for e in range(_D):
                    colvec = jnp.full((16,), e, jnp.int32)
                    vals = plsc.load_gather(rows[p], [rowvec, colvec])
                    slab[p][e // 8, pl.ds(gofs + (e % 8) * 128, 16)] = vals

        @pl.loop(0, _H // 2)
        def h_loop(i):
            for p in range(2):
                h = 2 * i + p
                gather(h, p).wait()

                @pl.when(i >= 1)
                def _():
                    for c in out_copies(h - 2, p):
                        c.wait()

                assemble(p)
                for c in out_copies(h, p):
                    c.start()

                @pl.when(i < _H // 2 - 1)
                def _():
                    gather(h + 2, p).start()

        for p in range(2):
            for c in out_copies(_H - 2 + p, p):
                c.wait()

    return k(w2d, xt)


def kernel(x, W):
    wt = W.T
    tail1d = W[_NFULL * _CA:].reshape(_TAIL * _D)
    w1d = _transpose_table(wt, tail1d)
    w2d = w1d.reshape(_V, _D)
    xt = x.T
    out3 = _gather_to_native(w2d, xt)
    out = (out3.reshape(_H, _D // 8, _B // 128, 8, 128)
           .transpose(2, 4, 0, 1, 3)
           .reshape(_B, _H, _D))
    return out


# speedup vs baseline: 2.0166x; 1.3670x over previous
import functools

import jax
import jax.numpy as jnp
from jax import lax
from jax.experimental import pallas as pl
from jax.experimental.pallas import tpu as pltpu
from jax.experimental.pallas import tpu_sc as plsc

_INFO = plsc.get_sparse_core_info()
_NC = _INFO.num_cores
_NS = _INFO.num_subcores
_NW = _NC * _NS

_V = 1000000
_D = 32
_CA = 512
_NFULL = _V // _CA
_TAIL = _V - _NFULL * _CA
_B = 16384
_H = 50
_CB = _B // _NW


def _iota16():
    return lax.iota(jnp.int32, 16)


@jax.jit
def _transpose_table(wt, tail1d):
    mesh = plsc.VectorSubcoreMesh(core_axis_name="c", subcore_axis_name="s")

    @functools.partial(
        pl.kernel,
        mesh=mesh,
        out_type=jax.ShapeDtypeStruct((_V * _D,), jnp.float32),
        scratch_types=[
            [pltpu.VMEM((_D, _CA), jnp.float32) for _ in range(2)],
            [pltpu.VMEM((_CA * _D,), jnp.float32) for _ in range(2)],
            [pltpu.SemaphoreType.DMA for _ in range(2)],
            [pltpu.SemaphoreType.DMA for _ in range(2)],
        ],
        compiler_params=pltpu.CompilerParams(needs_layout_passes=False),
    )
    def k(wt_hbm, tail_hbm, w1d_hbm, inb, outb, sem_i, sem_o):
        wid = lax.axis_index("s") * _NC + lax.axis_index("c")
        nw = (_NFULL - 1 - wid) // _NW + 1
        iota32 = _iota16() * _D

        def chunk_of(j):
            return wid + _NW * j

        def in_copy(j, p):
            v0 = chunk_of(j) * _CA
            return pltpu.make_async_copy(
                wt_hbm.at[:, pl.ds(v0, _CA)], inb[p], sem_i[p])

        def out_copy(j, p):
            o0 = chunk_of(j) * _CA * _D
            return pltpu.make_async_copy(
                outb[p], w1d_hbm.at[pl.ds(o0, _CA * _D)], sem_o[p])

        in_copy(0, 0).start()
        in_copy(1, 1).start()

        def transpose_chunk(p):
            @plsc.parallel_loop(0, _CA // 16, unroll=4)
            def g_loop(g):
                base = g * (16 * _D)
                for e in range(_D):
                    vals = inb[p][e, pl.ds(g * 16, 16)]
                    plsc.store_scatter(outb[p], [iota32 + (base + e)], vals)

        @pl.loop(0, 31)
        def j_loop(i):
            for p in range(2):
                j = 2 * i + p

                @pl.when(j < nw)
                def _():
                    in_copy(j, p).wait()

                    @pl.when(j >= 2)
                    def _():
                        out_copy(j - 2, p).wait()

                    transpose_chunk(p)
                    out_copy(j, p).start()

                    @pl.when(j + 2 < nw)
                    def _():
                        in_copy(j + 2, p).start()

        out_copy(nw - 2, 0).wait()
        out_copy(nw - 1, 1).wait()

        @pl.when(wid == 1)
        def _():
            pltpu.sync_copy(tail_hbm, outb[0].at[pl.ds(0, _TAIL * _D)])
            pltpu.sync_copy(outb[0].at[pl.ds(0, _TAIL * _D)],
                            w1d_hbm.at[pl.ds(_NFULL * _CA * _D, _TAIL * _D)])

    return k(wt, tail1d)


@jax.jit
def _gather_to_native(w2d, xt):
    mesh = plsc.VectorSubcoreMesh(core_axis_name="c", subcore_axis_name="s")

    @functools.partial(
        pl.kernel,
        mesh=mesh,
        out_type=jax.ShapeDtypeStruct((_H, _D // 8, (_B // 128) * 8 * 128),
                                      jnp.float32),
        scratch_types=[
            pltpu.VMEM((_H, _CB), jnp.int32),
            [pltpu.VMEM((_CB, _D), jnp.float32) for _ in range(2)],
            [pltpu.VMEM((_D // 8, (_CB // 128) * 8 * 128), jnp.float32)
             for _ in range(2)],
            [pltpu.SemaphoreType.DMA for _ in range(2)],
            [pltpu.SemaphoreType.DMA for _ in range(2)],
        ],
        compiler_params=pltpu.CompilerParams(use_tc_tiling_on_sc=False,
                                             needs_layout_passes=False),
    )
    def k(w_hbm, xt_hbm, out_hbm, idx_all, rows, slab, sem_g, sem_s):
        wid = lax.axis_index("s") * _NC + lax.axis_index("c")
        b0 = wid * _CB
        iota = _iota16()

        pltpu.sync_copy(xt_hbm.at[:, pl.ds(b0, _CB)], idx_all)

        def gather(h, p):
            return pltpu.make_async_copy(
                w_hbm.at[idx_all.at[h]], rows[p], sem_g[p])

        def out_copies(h, p):
            return [
                pltpu.make_async_copy(
                    slab[p].at[ti],
                    out_hbm.at[h, ti, pl.ds(wid * (_CB // 128) * 1024,
                                            (_CB // 128) * 1024)],
                    sem_s[p])
                for ti in range(_D // 8)
            ]

        gather(0, 0).start()
        gather(1, 1).start()

        def assemble(p):
            @plsc.parallel_loop(0, _CB // 16, unroll=4)
            def g_loop(g):
                rowvec = iota + g * 16
                gofs = (g // 8) * 1024 + (g % 8) * 16
                for e in range(_D):
                    colvec = jnp.full((16,), e, jnp.int32)
                    vals = plsc.load_gather(rows[p], [rowvec, colvec])
                    slab[p][e // 8, pl.ds(gofs + (e % 8) * 128, 16)] = vals

        @pl.loop(0, _H // 2)
        def h_loop(i):
            for p in range(2):
                h = 2 * i + p
                gather(h, p).wait()

                @pl.when(i >= 1)
                def _():
                    for c in out_copies(h - 2, p):
                        c.wait()

                assemble(p)
                for c in out_copies(h, p):
                    c.start()

                @pl.when(i < _H // 2 - 1)
                def _():
                    gather(h + 2, p).start()

        for p in range(2):
            for c in out_copies(_H - 2 + p, p):
                c.wait()

    return k(w2d, xt)


def kernel(x, W):
    wt = W.T
    tail1d = W[_NFULL * _CA:].reshape(_TAIL * _D)
    w1d = _transpose_table(wt, tail1d)
    w2d = w1d.reshape(_V, _D)
    xt = x.T
    out3 = _gather_to_native(w2d, xt)
    out = (out3.reshape(_H, _D // 8, _B // 128, 8, 128)
           .transpose(2, 4, 0, 1, 3)
           .reshape(_B, _H, _D))
    return out
